# Initial kernel scaffold; baseline (speedup 1.0000x reference)
import functools

import jax
import jax.numpy as jnp
from jax import lax
from jax.experimental import pallas as pl
from jax.experimental.pallas import tpu as pltpu
from jax.experimental.pallas import tpu_sc as plsc

F32 = jnp.float32


def _tc_pre_body(h2_ref, w1_ref, ats_ref, atd_ref, xp_ref, as_ref, ad_ref):
    x1 = jnp.dot(h2_ref[...], w1_ref[...], preferred_element_type=F32)
    as_ref[...] = jnp.sum(x1 * ats_ref[...], axis=1, keepdims=True)
    ad_ref[...] = jnp.sum(x1 * atd_ref[...], axis=1, keepdims=True)
    xp_ref[...] = jnp.stack([x1[:, :128], x1[:, 128:]], axis=0)


def _tc_pre(h2, W1, att_src1, att_dst1, n, blk):
    grid = n // blk
    return pl.pallas_call(
        _tc_pre_body,
        grid=(grid,),
        in_specs=[
            pl.BlockSpec((blk, 128), lambda i: (i, 0)),
            pl.BlockSpec((128, 256), lambda i: (0, 0)),
            pl.BlockSpec((1, 256), lambda i: (0, 0)),
            pl.BlockSpec((1, 256), lambda i: (0, 0)),
        ],
        out_specs=[
            pl.BlockSpec((2, blk, 128), lambda i: (0, i, 0)),
            pl.BlockSpec((blk, 1), lambda i: (i, 0)),
            pl.BlockSpec((blk, 1), lambda i: (i, 0)),
        ],
        out_shape=[
            jax.ShapeDtypeStruct((2, n, 128), F32),
            jax.ShapeDtypeStruct((n, 1), F32),
            jax.ShapeDtypeStruct((n, 1), F32),
        ],
    )(h2, W1, att_src1.reshape(1, 256), att_dst1.reshape(1, 256))


def _tc_mm_body(w2_ref, m_ref):
    m_ref[...] = lax.dot_general(
        w2_ref[...], w2_ref[...], (((1,), (1,)), ((), ())),
        preferred_element_type=F32)


def _tc_mm(W2):
    return pl.pallas_call(
        _tc_mm_body,
        out_shape=jax.ShapeDtypeStruct((256, 256), F32),
    )(W2)


def _elu(x):
    return jnp.where(x > 0, x, jnp.exp(jnp.minimum(x, 0.0)) - 1.0)


def _tc_mid_body(lo_ref, hi_ref, s_ref, w2_ref, m_ref, h2n_ref, x3_ref):
    acc = jnp.concatenate([lo_ref[...], hi_ref[...]], axis=1)
    s = s_ref[...][:, 0:1]
    h1 = _elu(acc / (s + 1e-16))
    h2n_ref[...] = jnp.dot(h1, w2_ref[...], preferred_element_type=F32)
    x3 = jnp.dot(h1, m_ref[...], preferred_element_type=F32)
    x3_ref[...] = jnp.stack([x3[:, :128], x3[:, 128:]], axis=0)


def _tc_mid(out1p, s16, W2, M, n, blk):
    grid = n // blk
    nb = n // blk
    return pl.pallas_call(
        _tc_mid_body,
        grid=(grid,),
        in_specs=[
            pl.BlockSpec((blk, 128), lambda i: (i, 0)),
            pl.BlockSpec((blk, 128), lambda i, _nb=nb: (i + _nb, 0)),
            pl.BlockSpec((blk, 16), lambda i: (i, 0)),
            pl.BlockSpec((256, 128), lambda i: (0, 0)),
            pl.BlockSpec((256, 256), lambda i: (0, 0)),
        ],
        out_specs=[
            pl.BlockSpec((blk, 128), lambda i: (i, 0)),
            pl.BlockSpec((2, blk, 128), lambda i: (0, i, 0)),
        ],
        out_shape=[
            jax.ShapeDtypeStruct((n, 128), F32),
            jax.ShapeDtypeStruct((2, n, 128), F32),
        ],
    )(out1p, out1p, s16, W2, M)


def _tc_post_body(lo_ref, hi_ref, s_ref, w4_ref, h4_ref):
    acc = jnp.concatenate([lo_ref[...], hi_ref[...]], axis=1)
    s = s_ref[...][:, 0:1]
    h3 = _elu(acc / (s + 1e-16))
    h4_ref[...] = jnp.dot(h3, w4_ref[...], preferred_element_type=F32)


def _tc_post(out3p, s16, W4, n, blk):
    grid = n // blk
    nb = n // blk
    return pl.pallas_call(
        _tc_post_body,
        grid=(grid,),
        in_specs=[
            pl.BlockSpec((blk, 128), lambda i: (i, 0)),
            pl.BlockSpec((blk, 128), lambda i, _nb=nb: (i + _nb, 0)),
            pl.BlockSpec((blk, 16), lambda i: (i, 0)),
            pl.BlockSpec((256, 512), lambda i: (0, 0)),
        ],
        out_specs=pl.BlockSpec((blk, 512), lambda i: (i, 0)),
        out_shape=jax.ShapeDtypeStruct((n, 512), F32),
    )(out3p, out3p, s16, W4)


_NC = 2
_NS = 16
_CH = 80


def _make_spmm(n, e, with_s):
    ept = e // _NS
    nchunks = ept // _CH
    zr = n // _NS
    assert e % _NS == 0 and ept % _CH == 0 and n % _NS == 0
    assert n % 80 == 0

    mesh = plsc.VectorSubcoreMesh(core_axis_name="c", subcore_axis_name="s")

    out_types = [jax.ShapeDtypeStruct((2 * n, 128), F32)]
    scratch = [
        pltpu.VMEM_SHARED((n, 128), F32),
        pltpu.VMEM((n,), F32),
        pltpu.VMEM((n,), F32),
        pltpu.VMEM((_CH,), jnp.int32),
        pltpu.VMEM((_CH,), jnp.int32),
        pltpu.VMEM((_CH,), jnp.int32),
        pltpu.VMEM((_CH, 128), F32),
        pltpu.VMEM((_CH,), F32),
        pltpu.SemaphoreType.DMA,
    ]
    if with_s:
        out_types.append(jax.ShapeDtypeStruct((n, 16), F32))
        scratch += [
            pltpu.VMEM_SHARED((n, 16), F32),
            pltpu.VMEM((_CH, 16), F32),
        ]
    sr = n // 10

    @functools.partial(pl.kernel, mesh=mesh, out_type=out_types,
                       scratch_types=scratch)
    def spmm(xp, srch, dsth, asrch, adsth, z2d, zs, *refs):
        if with_s:
            (outp, s_out, acc, asrc, adst, isrc, idst, ig, rows, ev, sem,
             s_sh, erows) = refs
        else:
            outp, acc, asrc, adst, isrc, idst, ig, rows, ev, sem = refs
        c = lax.axis_index("c")
        t = lax.axis_index("s")

        pltpu.sync_copy(z2d, acc.at[pl.ds(t * zr, zr)])
        if with_s:
            @pl.when(jnp.logical_and(c == 0, t < 10))
            def _zero_s():
                pltpu.sync_copy(zs, s_sh.at[pl.ds(t * sr, sr)])
        pltpu.sync_copy(asrch, asrc)
        pltpu.sync_copy(adsth, adst)
        plsc.subcore_barrier()

        lane = lax.broadcasted_iota(jnp.int32, (16,), 0)
        base0 = t * ept

        def chunk_body(k, _):
            base = base0 + k * _CH
            pltpu.sync_copy(srch.at[pl.ds(base, _CH)], isrc)
            pltpu.sync_copy(dsth.at[pl.ds(base, _CH)], idst)

            def grp(g, _):
                sv = isrc[pl.ds(g * 16, 16)]
                dv = idst[pl.ds(g * 16, 16)]
                al = (plsc.load_gather(asrc, [sv])
                      + plsc.load_gather(adst, [dv]))
                al = jnp.where(al >= 0, al, 0.2 * al)
                ev[pl.ds(g * 16, 16)] = jnp.exp(al)
                ig[pl.ds(g * 16, 16)] = sv + c * n
                return 0
            lax.fori_loop(0, _CH // 16, grp, 0)

            pltpu.async_copy(xp.at[ig], rows, sem).wait()

            def scale(kk, _):
                ekk = ev[kk]
                for f in range(8):
                    rows[kk, pl.ds(f * 16, 16)] = (
                        rows[kk, pl.ds(f * 16, 16)] * ekk)
                if with_s:
                    erows[kk, :] = jnp.where(lane == 0, ekk, 0.0)
                return 0
            lax.fori_loop(0, _CH, scale, 0)

            pltpu.sync_copy(rows, acc.at[idst], add=True)
            if with_s:
                @pl.when(c == 0)
                def _add_s():
                    pltpu.sync_copy(erows, s_sh.at[idst], add=True)
            return 0
        lax.fori_loop(0, nchunks, chunk_body, 0)

        plsc.subcore_barrier()
        pltpu.sync_copy(acc.at[pl.ds(t * zr, zr)],
                        outp.at[pl.ds(c * n + t * zr, zr)])
        if with_s:
            @pl.when(jnp.logical_and(c == 0, t < 10))
            def _flush_s():
                pltpu.sync_copy(s_sh.at[pl.ds(t * sr, sr)],
                                s_out.at[pl.ds(t * sr, sr)])
    return spmm


def kernel(h2, edge_index, W1, att_src1, att_dst1, W2, W4):
    n = h2.shape[0]
    e = edge_index.shape[1]
    blk = 1000
    src = edge_index[0]
    dst = edge_index[1]

    xp3, a_src, a_dst = _tc_pre(h2, W1, att_src1, att_dst1, n, blk)
    M = _tc_mm(W2)

    z2d = jnp.zeros((n // _NS, 128), F32)
    zs = jnp.zeros((n // 10, 16), F32)
    asrc = a_src.reshape(n)
    adst = a_dst.reshape(n)

    spmm_s = _make_spmm(n, e, with_s=True)
    spmm = _make_spmm(n, e, with_s=False)

    out1p, s16 = spmm_s(xp3.reshape(2 * n, 128), src, dst, asrc, adst,
                        z2d, zs)
    h2n, x3p3 = _tc_mid(out1p, s16, W2, M, n, blk)
    out3p = spmm(x3p3.reshape(2 * n, 128), src, dst, asrc, adst,
                 z2d, zs)
    if isinstance(out3p, (list, tuple)):
        out3p = out3p[0]
    h4 = _tc_post(out3p, s16, W4, n, blk)
    return (h2n, h4)


# speedup vs baseline: 8.3774x; 8.3774x over previous
import functools

import jax
import jax.numpy as jnp
from jax import lax
from jax.experimental import pallas as pl
from jax.experimental.pallas import tpu as pltpu
from jax.experimental.pallas import tpu_sc as plsc

F32 = jnp.float32


def _tc_pre_body(h2_ref, w1_ref, ats_ref, atd_ref, xp_ref, as_ref, ad_ref):
    x1 = jnp.dot(h2_ref[...], w1_ref[...], preferred_element_type=F32)
    as_ref[...] = jnp.sum(x1 * ats_ref[...], axis=1, keepdims=True)
    ad_ref[...] = jnp.sum(x1 * atd_ref[...], axis=1, keepdims=True)
    xp_ref[...] = jnp.stack([x1[:, :128], x1[:, 128:]], axis=0)


def _tc_pre(h2, W1, att_src1, att_dst1, n, blk):
    grid = n // blk
    return pl.pallas_call(
        _tc_pre_body,
        grid=(grid,),
        in_specs=[
            pl.BlockSpec((blk, 128), lambda i: (i, 0)),
            pl.BlockSpec((128, 256), lambda i: (0, 0)),
            pl.BlockSpec((1, 256), lambda i: (0, 0)),
            pl.BlockSpec((1, 256), lambda i: (0, 0)),
        ],
        out_specs=[
            pl.BlockSpec((2, blk, 128), lambda i: (0, i, 0)),
            pl.BlockSpec((blk, 1), lambda i: (i, 0)),
            pl.BlockSpec((blk, 1), lambda i: (i, 0)),
        ],
        out_shape=[
            jax.ShapeDtypeStruct((2, n, 128), F32),
            jax.ShapeDtypeStruct((n, 1), F32),
            jax.ShapeDtypeStruct((n, 1), F32),
        ],
    )(h2, W1, att_src1.reshape(1, 256), att_dst1.reshape(1, 256))


def _tc_mm_body(w2_ref, m_ref):
    m_ref[...] = lax.dot_general(
        w2_ref[...], w2_ref[...], (((1,), (1,)), ((), ())),
        preferred_element_type=F32)


def _tc_mm(W2):
    return pl.pallas_call(
        _tc_mm_body,
        out_shape=jax.ShapeDtypeStruct((256, 256), F32),
    )(W2)


def _elu(x):
    return jnp.where(x > 0, x, jnp.exp(jnp.minimum(x, 0.0)) - 1.0)


def _tc_mid_body(lo_ref, hi_ref, s_ref, w2_ref, m_ref, h2n_ref, x3_ref):
    acc = jnp.concatenate([lo_ref[...], hi_ref[...]], axis=1)
    s = lax.dot_general(s_ref[...], jnp.full((32, 1), 0.5, F32),
                        (((1,), (0,)), ((), ())), preferred_element_type=F32)
    h1 = _elu(acc / (s + 1e-16))
    h2n_ref[...] = jnp.dot(h1, w2_ref[...], preferred_element_type=F32)
    x3 = jnp.dot(h1, m_ref[...], preferred_element_type=F32)
    x3_ref[...] = jnp.stack([x3[:, :128], x3[:, 128:]], axis=0)


def _tc_mid(out1p, s16, W2, M, n, blk):
    grid = n // blk
    nb = n // blk
    return pl.pallas_call(
        _tc_mid_body,
        grid=(grid,),
        in_specs=[
            pl.BlockSpec((blk, 128), lambda i: (i, 0)),
            pl.BlockSpec((blk, 128), lambda i, _nb=nb: (i + _nb, 0)),
            pl.BlockSpec((blk, 32), lambda i: (i, 0)),
            pl.BlockSpec((256, 128), lambda i: (0, 0)),
            pl.BlockSpec((256, 256), lambda i: (0, 0)),
        ],
        out_specs=[
            pl.BlockSpec((blk, 128), lambda i: (i, 0)),
            pl.BlockSpec((2, blk, 128), lambda i: (0, i, 0)),
        ],
        out_shape=[
            jax.ShapeDtypeStruct((n, 128), F32),
            jax.ShapeDtypeStruct((2, n, 128), F32),
        ],
    )(out1p, out1p, s16, W2, M)


def _tc_post_body(lo_ref, hi_ref, s_ref, w4_ref, h4_ref):
    acc = jnp.concatenate([lo_ref[...], hi_ref[...]], axis=1)
    s = lax.dot_general(s_ref[...], jnp.full((32, 1), 0.5, F32),
                        (((1,), (0,)), ((), ())), preferred_element_type=F32)
    h3 = _elu(acc / (s + 1e-16))
    h4_ref[...] = jnp.dot(h3, w4_ref[...], preferred_element_type=F32)


def _tc_post(out3p, s16, W4, n, blk):
    grid = n // blk
    nb = n // blk
    return pl.pallas_call(
        _tc_post_body,
        grid=(grid,),
        in_specs=[
            pl.BlockSpec((blk, 128), lambda i: (i, 0)),
            pl.BlockSpec((blk, 128), lambda i, _nb=nb: (i + _nb, 0)),
            pl.BlockSpec((blk, 32), lambda i: (i, 0)),
            pl.BlockSpec((256, 512), lambda i: (0, 0)),
        ],
        out_specs=pl.BlockSpec((blk, 512), lambda i: (i, 0)),
        out_shape=jax.ShapeDtypeStruct((n, 512), F32),
    )(out3p, out3p, s16, W4)


_NC = 2
_NS = 16
_CH = 80


def _make_spmm(n, e, with_s):
    ch = _CH
    ept = e // _NS
    nchunks = ept // ch
    zr = n // 10
    assert e % _NS == 0 and ept % ch == 0
    assert n % 80 == 0 and zr % 8 == 0

    mesh = plsc.VectorSubcoreMesh(core_axis_name="c", subcore_axis_name="s")

    out_types = [jax.ShapeDtypeStruct((2 * n, 128), F32)]
    scratch = [
        pltpu.VMEM_SHARED((n, 128), F32),
        pltpu.VMEM((n,), F32),
        pltpu.VMEM((n,), F32),
        pltpu.VMEM((ch,), jnp.int32),
        pltpu.VMEM((ch,), jnp.int32),
        pltpu.VMEM((ch,), jnp.int32),
        pltpu.VMEM((ch, 128), F32),
        pltpu.VMEM((ch,), F32),
        pltpu.VMEM((40, 128), F32),
        pltpu.SemaphoreType.DMA,
    ]
    if with_s:
        out_types.append(jax.ShapeDtypeStruct((2 * _NS, n), F32))
        scratch += [
            pltpu.VMEM((n,), F32),
        ]

    @functools.partial(
        pl.kernel, mesh=mesh, out_type=out_types, scratch_types=scratch,
        compiler_params=pltpu.CompilerParams(needs_layout_passes=False))
    def spmm(xp, srch, dsth, asrch, adsth, *refs):
        if with_s:
            (outp, s_out, acc, asrc, adst, isrc, idst, ig, rows, ev, zbuf,
             sem, s_part) = refs
        else:
            outp, acc, asrc, adst, isrc, idst, ig, rows, ev, zbuf, sem = refs
        c = lax.axis_index("c")
        t = lax.axis_index("s")
        zero16 = jnp.zeros((16,), F32)

        def _zrow(i, _):
            for f in range(8):
                zbuf[i, pl.ds(f * 16, 16)] = zero16
            return 0
        lax.fori_loop(0, 40, _zrow, 0)
        if with_s:
            def _zs(i, _):
                s_part[pl.ds(i * 16, 16)] = zero16
                return 0
            lax.fori_loop(0, n // 16, _zs, 0)

        @pl.when(t < 10)
        def _zero_acc():
            for j in range(zr // 40):
                pltpu.sync_copy(zbuf, acc.at[pl.ds(t * zr + j * 40, 40)])

        pltpu.sync_copy(asrch, asrc)
        pltpu.sync_copy(adsth, adst)
        plsc.subcore_barrier()

        lane = lax.broadcasted_iota(jnp.int32, (16,), 0)
        base0 = t * ept

        def chunk_body(k, _):
            base = base0 + k * ch
            pltpu.sync_copy(srch.at[pl.ds(base, ch)], isrc)
            pltpu.sync_copy(dsth.at[pl.ds(base, ch)], idst)

            def grp(g, _):
                sv = isrc[pl.ds(g * 16, 16)]
                dv = idst[pl.ds(g * 16, 16)]
                al = (plsc.load_gather(asrc, [sv])
                      + plsc.load_gather(adst, [dv]))
                al = jnp.where(al >= 0, al, 0.2 * al)
                evv = jnp.exp(al)
                ev[pl.ds(g * 16, 16)] = evv
                ig[pl.ds(g * 16, 16)] = sv + c * n
                if with_s:
                    plsc.addupdate_scatter(s_part, [dv], evv)
                return 0
            lax.fori_loop(0, ch // 16, grp, 0)

            pltpu.async_copy(xp.at[ig], rows, sem).wait()

            def scale(g, _):
                evg = ev[pl.ds(g * 16, 16)]
                for j in range(16):
                    ek = evg[j]
                    kk = g * 16 + j
                    for f in range(8):
                        rows[kk, pl.ds(f * 16, 16)] = (
                            rows[kk, pl.ds(f * 16, 16)] * ek)
                return 0
            lax.fori_loop(0, ch // 16, scale, 0)

            pltpu.sync_copy(rows, acc.at[idst], add=True)
            return 0
        lax.fori_loop(0, nchunks, chunk_body, 0)

        plsc.subcore_barrier()

        @pl.when(t < 10)
        def _flush_acc():
            pltpu.sync_copy(acc.at[pl.ds(t * zr, zr)],
                            outp.at[pl.ds(c * n + t * zr, zr)])
        if with_s:
            pltpu.sync_copy(s_part, s_out.at[c * _NS + t])
    return spmm


def kernel(h2, edge_index, W1, att_src1, att_dst1, W2, W4):
    n = h2.shape[0]
    e = edge_index.shape[1]
    blk = 1000
    src = edge_index[0]
    dst = edge_index[1]

    xp3, a_src, a_dst = _tc_pre(h2, W1, att_src1, att_dst1, n, blk)
    M = _tc_mm(W2)

    asrc = a_src.reshape(n)
    adst = a_dst.reshape(n)

    spmm_s = _make_spmm(n, e, with_s=True)
    spmm = _make_spmm(n, e, with_s=False)

    out1p, s_parts = spmm_s(xp3.reshape(2 * n, 128), src, dst, asrc, adst)
    s16 = s_parts.T
    h2n, x3p3 = _tc_mid(out1p, s16, W2, M, n, blk)
    out3p = spmm(x3p3.reshape(2 * n, 128), src, dst, asrc, adst)
    if isinstance(out3p, (list, tuple)):
        out3p = out3p[0]
    h4 = _tc_post(out3p, s16, W4, n, blk)
    return (h2n, h4)
